# baseline (speedup 1.0000x reference)
import functools

import jax
import jax.numpy as jnp
from jax.experimental import pallas as pl


def _mm_body(a_ref, b_ref, o_ref):
    o_ref[...] = jnp.dot(a_ref[...], b_ref[...],
                         preferred_element_type=jnp.float32)


def _mm_bias_body(a_ref, b_ref, bias_ref, o_ref, *, relu):
    acc = jnp.dot(a_ref[...], b_ref[...], preferred_element_type=jnp.float32)
    acc = acc + bias_ref[...]
    if relu:
        acc = jnp.maximum(acc, 0.0)
    o_ref[...] = acc


def _matmul(a, b, bias=None, relu=False, block_m=512):
    m, k = a.shape
    k2, n = b.shape
    assert k == k2
    grid = (pl.cdiv(m, block_m),)
    in_specs = [
        pl.BlockSpec((block_m, k), lambda i: (i, 0)),
        pl.BlockSpec((k, n), lambda i: (0, 0)),
    ]
    operands = [a, b]
    if bias is not None:
        in_specs.append(pl.BlockSpec((1, n), lambda i: (0, 0)))
        operands.append(bias.reshape(1, n))
        body = functools.partial(_mm_bias_body, relu=relu)
    else:
        body = _mm_body
    return pl.pallas_call(
        body,
        grid=grid,
        in_specs=in_specs,
        out_specs=pl.BlockSpec((block_m, n), lambda i: (i, 0)),
        out_shape=jax.ShapeDtypeStruct((m, n), jnp.float32),
    )(*operands)


def _gcn(h, adj, w, b, relu):
    support = _matmul(h, w)
    return _matmul(adj, support, bias=b, relu=relu)


def kernel(x, adj, We1, be1, We2, be2, We3, be3, Wez, bez,
           Wd1, bd1, Wd2, bd2, Wd3, bd3, Wdf, bdf):
    h = x
    for w, b in ((We1, be1), (We2, be2), (We3, be3)):
        h = _gcn(h, adj, w, b, relu=True)
    z = _gcn(h, adj, Wez, bez, relu=False)
    h = z
    for w, b in ((Wd1, bd1), (Wd2, bd2), (Wd3, bd3)):
        h = _gcn(h, adj, w, b, relu=True)
    x_recon = _gcn(h, adj, Wdf, bdf, relu=False)
    return (z, x_recon)


# speedup vs baseline: 1.1431x; 1.1431x over previous
import functools

import jax
import jax.numpy as jnp
from jax.experimental import pallas as pl
from jax.experimental.pallas import tpu as pltpu


def _gcn_body(h_ref, w_ref, adj_ref, bias_ref, o_ref, s_ref, *, relu):
    @pl.when(pl.program_id(0) == 0)
    def _():
        s_ref[...] = jnp.dot(h_ref[...], w_ref[...],
                             preferred_element_type=jnp.float32)

    acc = jnp.dot(adj_ref[...], s_ref[...],
                  preferred_element_type=jnp.float32)
    acc = acc + bias_ref[...]
    if relu:
        acc = jnp.maximum(acc, 0.0)
    o_ref[...] = acc


def _gcn(h, adj, w, b, relu, block_m=512):
    m, k = h.shape
    k2, n = w.shape
    grid = (pl.cdiv(m, block_m),)
    return pl.pallas_call(
        functools.partial(_gcn_body, relu=relu),
        grid=grid,
        in_specs=[
            pl.BlockSpec((m, k), lambda i: (0, 0)),
            pl.BlockSpec((k, n), lambda i: (0, 0)),
            pl.BlockSpec((block_m, m), lambda i: (i, 0)),
            pl.BlockSpec((1, n), lambda i: (0, 0)),
        ],
        out_specs=pl.BlockSpec((block_m, n), lambda i: (i, 0)),
        out_shape=jax.ShapeDtypeStruct((m, n), jnp.float32),
        scratch_shapes=[pltpu.VMEM((m, n), jnp.float32)],
    )(h, w, adj, b.reshape(1, n))


def kernel(x, adj, We1, be1, We2, be2, We3, be3, Wez, bez,
           Wd1, bd1, Wd2, bd2, Wd3, bd3, Wdf, bdf):
    h = x
    for w, b in ((We1, be1), (We2, be2), (We3, be3)):
        h = _gcn(h, adj, w, b, relu=True)
    z = _gcn(h, adj, Wez, bez, relu=False)
    h = z
    for w, b in ((Wd1, bd1), (Wd2, bd2), (Wd3, bd3)):
        h = _gcn(h, adj, w, b, relu=True)
    x_recon = _gcn(h, adj, Wdf, bdf, relu=False)
    return (z, x_recon)
